# Initial kernel scaffold; baseline (speedup 1.0000x reference)
import jax
import jax.numpy as jnp
from jax.experimental import pallas as pl


def kernel(x, grid1_table, grid0_table):
    raise NotImplementedError("write your pallas kernel here")


# speedup vs baseline: 70.4874x; 70.4874x over previous
import functools

import jax
import jax.numpy as jnp
from jax import lax
from jax.experimental import pallas as pl
from jax.experimental.pallas import tpu as pltpu
from jax.experimental.pallas import tpu_sc as plsc

N_PTS = 4194304
RES_UP = 688
RES_DN = 20
PATCH_R = (RES_UP - 1) * (RES_UP - 1)

NW = 32
PW = N_PTS // NW
B = 1024
NCHUNK = PW // B
GRP = B // 128
NVREG = B // 16


def _sigmoid_body(x_ref, o_ref):
    o_ref[...] = 1.0 / (1.0 + jnp.exp(-x_ref[...]))


def _sigmoid_tables(grid1_table, grid0_table):
    g1 = grid1_table.reshape(-1)
    g0 = grid0_table.reshape(-1)
    n1 = g1.shape[0]
    n0 = g0.shape[0]
    pad = (-(n1 + n0)) % 128
    flat = jnp.concatenate([g1, g0, jnp.zeros((pad,), jnp.float32)])
    flat2 = flat.reshape(-1, 128)
    sig = pl.pallas_call(
        _sigmoid_body,
        out_shape=jax.ShapeDtypeStruct(flat2.shape, jnp.float32),
    )(flat2).reshape(-1)
    sig1 = sig[:n1].reshape(RES_UP, RES_UP, 3)
    sig0 = sig[n1:n1 + n0].reshape(RES_DN * RES_DN * RES_DN, 3)
    return sig1, sig0


mesh = plsc.VectorSubcoreMesh(core_axis_name="c", subcore_axis_name="s")


@functools.partial(
    pl.kernel,
    mesh=mesh,
    compiler_params=pltpu.CompilerParams(
        needs_layout_passes=False, use_tc_tiling_on_sc=False
    ),
    out_type=jax.ShapeDtypeStruct((N_PTS, 3), jnp.float32),
    scratch_types=[
        pltpu.VMEM((2 * B,), jnp.float32),
        pltpu.VMEM((B,), jnp.int32),
        pltpu.VMEM((B,), jnp.float32),
        pltpu.VMEM((B,), jnp.float32),
        pltpu.VMEM((B, 16), jnp.float32),
        pltpu.VMEM((RES_DN ** 3, 4), jnp.float32),
        pltpu.VMEM((B, 3), jnp.float32),
        pltpu.SemaphoreType.DMA,
    ],
)
def _sc_lookup(x_hbm, patch_hbm, sig0_hbm, out_hbm,
               x_v, idx_v, fu_v, fv_v, cor_v, s0_v, out_v, gsem):
    wid = lax.axis_index("s") * 2 + lax.axis_index("c")
    base0 = wid * PW
    pltpu.sync_copy(sig0_hbm, s0_v)
    lanes = lax.iota(jnp.int32, 16)

    def const16(val):
        return jnp.full((16,), val, jnp.int32)

    def chunk_body(t, carry):
        base = base0 + t * B
        pltpu.sync_copy(x_hbm.at[pl.ds(2 * base, 2 * B)], x_v)

        def p1(j, c1):
            rows = j * 16 + lanes
            xu = plsc.load_gather(x_v, [rows * 2])
            xv = plsc.load_gather(x_v, [rows * 2 + 1])
            u = jnp.minimum(jnp.maximum(xu, 0.0), 1.0) * float(RES_UP - 1)
            v = jnp.minimum(jnp.maximum(xv, 0.0), 1.0) * float(RES_UP - 1)
            ui = jnp.minimum(u.astype(jnp.int32), RES_UP - 2)
            vi = jnp.minimum(v.astype(jnp.int32), RES_UP - 2)
            idx_v[pl.ds(j * 16, 16)] = ui * (RES_UP - 1) + vi
            fu_v[pl.ds(j * 16, 16)] = u - ui.astype(jnp.float32)
            fv_v[pl.ds(j * 16, 16)] = v - vi.astype(jnp.float32)
            return c1

        lax.fori_loop(0, NVREG, p1, 0)

        copies = [
            pltpu.async_copy(
                patch_hbm.at[idx_v.at[pl.ds(k * 128, 128)]],
                cor_v.at[pl.ds(k * 128, 128)],
                gsem,
            )
            for k in range(GRP)
        ]
        for cp in copies:
            cp.wait()

        def p2(j, c2):
            rows = j * 16 + lanes
            fu = fu_v[pl.ds(j * 16, 16)]
            fv = fv_v[pl.ds(j * 16, 16)]
            w00 = (1.0 - fu) * (1.0 - fv)
            w01 = (1.0 - fu) * fv
            w10 = fu * (1.0 - fv)
            w11 = fu * fv
            key = []
            for ch in range(3):
                c00 = plsc.load_gather(cor_v, [rows, const16(ch)])
                c01 = plsc.load_gather(cor_v, [rows, const16(3 + ch)])
                c10 = plsc.load_gather(cor_v, [rows, const16(6 + ch)])
                c11 = plsc.load_gather(cor_v, [rows, const16(9 + ch)])
                key.append(w00 * c00 + w01 * c01 + w10 * c10 + w11 * c11)

            s = key[0] * float(RES_DN - 1)
            xi = jnp.minimum(s.astype(jnp.int32), RES_DN - 2)
            fx = s - xi.astype(jnp.float32)
            s = key[1] * float(RES_DN - 1)
            yi = jnp.minimum(s.astype(jnp.int32), RES_DN - 2)
            fy = s - yi.astype(jnp.float32)
            s = key[2] * float(RES_DN - 1)
            zi = jnp.minimum(s.astype(jnp.int32), RES_DN - 2)
            fz = s - zi.astype(jnp.float32)

            cell = (xi * RES_DN + yi) * RES_DN + zi
            p00 = (1.0 - fx) * (1.0 - fy)
            p01 = (1.0 - fx) * fy
            p10 = fx * (1.0 - fy)
            p11 = fx * fy
            gz0 = 1.0 - fz
            w8 = [p00 * gz0, p00 * fz, p01 * gz0, p01 * fz,
                  p10 * gz0, p10 * fz, p11 * gz0, p11 * fz]
            offs = [0, 1, RES_DN, RES_DN + 1,
                    RES_DN * RES_DN, RES_DN * RES_DN + 1,
                    RES_DN * RES_DN + RES_DN, RES_DN * RES_DN + RES_DN + 1]
            for ch in range(3):
                acc = w8[0] * plsc.load_gather(s0_v, [cell, const16(ch)])
                for kk in range(1, 8):
                    val = plsc.load_gather(s0_v, [cell + offs[kk], const16(ch)])
                    acc = acc + w8[kk] * val
                plsc.store_scatter(out_v, [rows, const16(ch)], acc)
            return c2

        lax.fori_loop(0, NVREG, p2, 0)
        pltpu.sync_copy(out_v, out_hbm.at[pl.ds(base, B)])
        return carry

    lax.fori_loop(0, NCHUNK, chunk_body, 0)


def kernel(x, grid1_table, grid0_table):
    sig1, sig0 = _sigmoid_tables(grid1_table, grid0_table)
    r = RES_UP - 1
    a = sig1[:r, :r, :]
    b = sig1[:r, 1:, :]
    c = sig1[1:, :r, :]
    d = sig1[1:, 1:, :]
    patch = jnp.concatenate(
        [a, b, c, d, jnp.zeros((r, r, 4), jnp.float32)], axis=-1
    ).reshape(PATCH_R, 16)
    sig0p = jnp.concatenate(
        [sig0, jnp.zeros((RES_DN ** 3, 1), jnp.float32)], axis=1
    )
    return _sc_lookup(x.reshape(-1), patch, sig0p)


# speedup vs baseline: 71.5581x; 1.0152x over previous
import functools

import jax
import jax.numpy as jnp
from jax import lax
from jax.experimental import pallas as pl
from jax.experimental.pallas import tpu as pltpu
from jax.experimental.pallas import tpu_sc as plsc

N_PTS = 4194304
RES_UP = 688
RES_DN = 20
PR = RES_UP - 1
PATCH_R = PR * PR
ROWW = RES_UP * 3
N3 = RES_DN * RES_DN * RES_DN

NW = 32
PW = N_PTS // NW
B = 1024
NCHUNK = PW // B
GRP = B // 128
NVREG = B // 16

UROWS_PER_W = (PR + NW - 1) // NW

_SC_PARAMS = pltpu.CompilerParams(
    needs_layout_passes=False, use_tc_tiling_on_sc=False
)

mesh = plsc.VectorSubcoreMesh(core_axis_name="c", subcore_axis_name="s")


def _sigmoid(v):
    return 1.0 / (1.0 + jnp.exp(-v))


@functools.partial(
    pl.kernel,
    mesh=mesh,
    compiler_params=_SC_PARAMS,
    out_type=(
        jax.ShapeDtypeStruct((PATCH_R, 16), jnp.float32),
        jax.ShapeDtypeStruct((N3, 4), jnp.float32),
    ),
    scratch_types=[
        pltpu.VMEM((2 * ROWW,), jnp.float32),
        pltpu.VMEM((2 * ROWW,), jnp.float32),
        pltpu.VMEM((PR, 16), jnp.float32),
        pltpu.VMEM((N3 * 3,), jnp.float32),
        pltpu.VMEM((N3, 4), jnp.float32),
        pltpu.SemaphoreType.DMA,
    ],
)
def _sc_build(g1_hbm, g0_hbm, patch_hbm, sig0_hbm,
              raw_v, sg_v, slab_v, g0_v, s0_v, sem):
    wid = lax.axis_index("s") * 2 + lax.axis_index("c")
    lanes = lax.iota(jnp.int32, 16)
    t0 = jnp.where(lanes < 6, lanes,
                   jnp.where(lanes < 12, ROWW + lanes - 6, 0))

    def urow(i, c0):
        u = wid * UROWS_PER_W + i

        @pl.when(u < PR)
        def _():
            pltpu.sync_copy(g1_hbm.at[pl.ds(u * ROWW, 2 * ROWW)], raw_v)

            def sig_body(s, c1):
                sg_v[pl.ds(s * 16, 16)] = _sigmoid(raw_v[pl.ds(s * 16, 16)])
                return c1

            lax.fori_loop(0, 2 * ROWW // 16, sig_body, 0)

            def cell(v, c2):
                vec = plsc.load_gather(sg_v, [t0 + 3 * v])
                plsc.store_scatter(slab_v, [jnp.full((16,), v, jnp.int32),
                                            lanes], vec)
                return c2

            lax.fori_loop(0, PR, cell, 0)
            pltpu.sync_copy(slab_v, patch_hbm.at[pl.ds(u * PR, PR)])

        return c0

    lax.fori_loop(0, UROWS_PER_W, urow, 0)

    @pl.when(wid == 0)
    def _():
        pltpu.sync_copy(g0_hbm, g0_v)
        t1 = jnp.where(lanes % 4 < 3, (lanes // 4) * 3 + lanes % 4, 0)
        msk = (lanes % 4 < 3).astype(jnp.float32)

        def g0_body(j, c3):
            vec = plsc.load_gather(g0_v, [t1 + 12 * j])
            s = _sigmoid(vec) * msk
            plsc.store_scatter(s0_v, [4 * j + lanes // 4, lanes % 4], s)
            return c3

        lax.fori_loop(0, N3 // 4, g0_body, 0)
        pltpu.sync_copy(s0_v, sig0_hbm)


@functools.partial(
    pl.kernel,
    mesh=mesh,
    compiler_params=_SC_PARAMS,
    out_type=jax.ShapeDtypeStruct((3 * N_PTS,), jnp.float32),
    scratch_types=[
        pltpu.VMEM((2 * B,), jnp.float32),
        pltpu.VMEM((B,), jnp.int32),
        pltpu.VMEM((B,), jnp.float32),
        pltpu.VMEM((B,), jnp.float32),
        pltpu.VMEM((B, 16), jnp.float32),
        pltpu.VMEM((N3, 4), jnp.float32),
        pltpu.VMEM((3 * B,), jnp.float32),
        pltpu.SemaphoreType.DMA,
    ],
)
def _sc_lookup(x_hbm, patch_hbm, sig0_hbm, out_hbm,
               x_v, idx_v, fu_v, fv_v, cor_v, s0_v, out_v, gsem):
    wid = lax.axis_index("s") * 2 + lax.axis_index("c")
    base0 = wid * PW
    pltpu.sync_copy(sig0_hbm, s0_v)
    lanes = lax.iota(jnp.int32, 16)

    def const16(val):
        return jnp.full((16,), val, jnp.int32)

    def chunk_body(t, carry):
        base = base0 + t * B
        pltpu.sync_copy(x_hbm.at[pl.ds(2 * base, 2 * B)], x_v)

        def p1(j, c1):
            rows = j * 16 + lanes
            xu = plsc.load_gather(x_v, [rows * 2])
            xv = plsc.load_gather(x_v, [rows * 2 + 1])
            u = jnp.minimum(jnp.maximum(xu, 0.0), 1.0) * float(RES_UP - 1)
            v = jnp.minimum(jnp.maximum(xv, 0.0), 1.0) * float(RES_UP - 1)
            ui = jnp.minimum(u.astype(jnp.int32), RES_UP - 2)
            vi = jnp.minimum(v.astype(jnp.int32), RES_UP - 2)
            idx_v[pl.ds(j * 16, 16)] = ui * PR + vi
            fu_v[pl.ds(j * 16, 16)] = u - ui.astype(jnp.float32)
            fv_v[pl.ds(j * 16, 16)] = v - vi.astype(jnp.float32)
            return c1

        lax.fori_loop(0, NVREG, p1, 0)

        copies = [
            pltpu.async_copy(
                patch_hbm.at[idx_v.at[pl.ds(k * 128, 128)]],
                cor_v.at[pl.ds(k * 128, 128)],
                gsem,
            )
            for k in range(GRP)
        ]
        for cp in copies:
            cp.wait()

        def p2(j, c2):
            rows = j * 16 + lanes
            fu = fu_v[pl.ds(j * 16, 16)]
            fv = fv_v[pl.ds(j * 16, 16)]
            w00 = (1.0 - fu) * (1.0 - fv)
            w01 = (1.0 - fu) * fv
            w10 = fu * (1.0 - fv)
            w11 = fu * fv
            key = []
            for ch in range(3):
                c00 = plsc.load_gather(cor_v, [rows, const16(ch)])
                c01 = plsc.load_gather(cor_v, [rows, const16(3 + ch)])
                c10 = plsc.load_gather(cor_v, [rows, const16(6 + ch)])
                c11 = plsc.load_gather(cor_v, [rows, const16(9 + ch)])
                key.append(w00 * c00 + w01 * c01 + w10 * c10 + w11 * c11)

            s = key[0] * float(RES_DN - 1)
            xi = jnp.minimum(s.astype(jnp.int32), RES_DN - 2)
            fx = s - xi.astype(jnp.float32)
            s = key[1] * float(RES_DN - 1)
            yi = jnp.minimum(s.astype(jnp.int32), RES_DN - 2)
            fy = s - yi.astype(jnp.float32)
            s = key[2] * float(RES_DN - 1)
            zi = jnp.minimum(s.astype(jnp.int32), RES_DN - 2)
            fz = s - zi.astype(jnp.float32)

            cell = (xi * RES_DN + yi) * RES_DN + zi
            p00 = (1.0 - fx) * (1.0 - fy)
            p01 = (1.0 - fx) * fy
            p10 = fx * (1.0 - fy)
            p11 = fx * fy
            gz0 = 1.0 - fz
            w8 = [p00 * gz0, p00 * fz, p01 * gz0, p01 * fz,
                  p10 * gz0, p10 * fz, p11 * gz0, p11 * fz]
            offs = [0, 1, RES_DN, RES_DN + 1,
                    RES_DN * RES_DN, RES_DN * RES_DN + 1,
                    RES_DN * RES_DN + RES_DN, RES_DN * RES_DN + RES_DN + 1]
            for ch in range(3):
                acc = w8[0] * plsc.load_gather(s0_v, [cell, const16(ch)])
                for kk in range(1, 8):
                    val = plsc.load_gather(s0_v, [cell + offs[kk], const16(ch)])
                    acc = acc + w8[kk] * val
                plsc.store_scatter(out_v, [rows * 3 + ch], acc)
            return c2

        lax.fori_loop(0, NVREG, p2, 0)
        pltpu.sync_copy(out_v, out_hbm.at[pl.ds(3 * base, 3 * B)])
        return carry

    lax.fori_loop(0, NCHUNK, chunk_body, 0)


def kernel(x, grid1_table, grid0_table):
    patch, sig0p = _sc_build(grid1_table.reshape(-1), grid0_table.reshape(-1))
    out = _sc_lookup(x.reshape(-1), patch, sig0p)
    return out.reshape(N_PTS, 3)


# speedup vs baseline: 264.7529x; 3.6998x over previous
import functools

import jax
import jax.numpy as jnp
from jax import lax
from jax.experimental import pallas as pl
from jax.experimental.pallas import tpu as pltpu
from jax.experimental.pallas import tpu_sc as plsc

N_PTS = 4194304
RES_UP = 688
RES_DN = 20
PR = RES_UP - 1
PATCH_R = PR * PR
ROWW = RES_UP * 3
N3 = RES_DN * RES_DN * RES_DN
G0W = N3 * 3
G0CH = 2000
NG0CH = G0W // G0CH

NW = 32
NT = 16
PW = N_PTS // NW
B = 1024
NCHUNK = PW // B
NVREG = B // 16

UROWS_PER_T = (PR + NT - 1) // NT

_SC_PARAMS = pltpu.CompilerParams(
    needs_layout_passes=False, use_tc_tiling_on_sc=False
)

mesh = plsc.VectorSubcoreMesh(core_axis_name="c", subcore_axis_name="s")


def _sigmoid(v):
    return 1.0 / (1.0 + jnp.exp(-v))


@functools.partial(
    pl.kernel,
    mesh=mesh,
    compiler_params=_SC_PARAMS,
    out_type=(
        jax.ShapeDtypeStruct((4 * N_PTS,), jnp.float32),
        jax.ShapeDtypeStruct((PATCH_R, 16), jnp.float32),
        jax.ShapeDtypeStruct((PATCH_R, 16), jnp.float32),
    ),
    scratch_types=[
        pltpu.VMEM((2 * ROWW,), jnp.float32),
        pltpu.VMEM((2 * ROWW,), jnp.float32),
        pltpu.VMEM((PR, 16), jnp.float32),
        pltpu.VMEM((G0W,), jnp.float32),
        pltpu.VMEM((2 * B,), jnp.float32),
        pltpu.VMEM((B,), jnp.int32),
        pltpu.VMEM((B,), jnp.float32),
        pltpu.VMEM((B,), jnp.float32),
        pltpu.VMEM((B, 16), jnp.float32),
        pltpu.VMEM((4 * B,), jnp.float32),
        pltpu.SemaphoreType.DMA,
    ],
)
def _sc_net(x_hbm, g1_hbm, g0_hbm, out_hbm, patch0_hbm, patch1_hbm,
            raw_v, sg_v, slab_v, s0_v,
            x_v, idx_v, fu_v, fv_v, cor_v, out_v, gsem):
    cid = lax.axis_index("c")
    sid = lax.axis_index("s")
    wid = sid * 2 + cid
    lanes = lax.iota(jnp.int32, 16)
    t0 = jnp.where(lanes < 6, lanes,
                   jnp.where(lanes < 12, ROWW + lanes - 6, 0))

    def urow(i, c0):
        u = sid * UROWS_PER_T + i

        @pl.when(u < PR)
        def _():
            pltpu.sync_copy(g1_hbm.at[pl.ds(u * ROWW, 2 * ROWW)], raw_v)

            def sig_body(s, c1):
                sg_v[pl.ds(s * 16, 16)] = _sigmoid(raw_v[pl.ds(s * 16, 16)])
                return c1

            lax.fori_loop(0, 2 * ROWW // 16, sig_body, 0)

            def cell(v, c2):
                vec = plsc.load_gather(sg_v, [t0 + 3 * v])
                plsc.store_scatter(slab_v, [jnp.full((16,), v, jnp.int32),
                                            lanes], vec)
                return c2

            lax.fori_loop(0, PR, cell, 0)

            @pl.when(cid == 0)
            def _():
                pltpu.sync_copy(slab_v, patch0_hbm.at[pl.ds(u * PR, PR)])

            @pl.when(cid == 1)
            def _():
                pltpu.sync_copy(slab_v, patch1_hbm.at[pl.ds(u * PR, PR)])

        return c0

    lax.fori_loop(0, UROWS_PER_T, urow, 0)

    def build0(p, c0):
        pltpu.sync_copy(g0_hbm.at[pl.ds(p * G0CH, G0CH)],
                        raw_v.at[pl.ds(0, G0CH)])

        def vr(m, c1):
            s0_v[pl.ds(p * G0CH + m * 16, 16)] = _sigmoid(
                raw_v[pl.ds(m * 16, 16)])
            return c1

        lax.fori_loop(0, G0CH // 16, vr, 0)
        return c0

    lax.fori_loop(0, NG0CH, build0, 0)

    plsc.subcore_barrier()

    base0 = wid * PW

    def const16(val):
        return jnp.full((16,), val, jnp.int32)

    def chunk_body(t, carry):
        base = base0 + t * B
        pltpu.sync_copy(x_hbm.at[pl.ds(2 * base, 2 * B)], x_v)

        def p1(j, c1):
            rows = j * 16 + lanes
            xu_at = (rows >> 7) * 256 + (rows & 127)
            xu = plsc.load_gather(x_v, [xu_at])
            xv = plsc.load_gather(x_v, [xu_at + 128])
            u = jnp.minimum(jnp.maximum(xu, 0.0), 1.0) * float(RES_UP - 1)
            v = jnp.minimum(jnp.maximum(xv, 0.0), 1.0) * float(RES_UP - 1)
            ui = jnp.minimum(u.astype(jnp.int32), RES_UP - 2)
            vi = jnp.minimum(v.astype(jnp.int32), RES_UP - 2)
            idx_v[pl.ds(j * 16, 16)] = ui * PR + vi
            fu_v[pl.ds(j * 16, 16)] = u - ui.astype(jnp.float32)
            fv_v[pl.ds(j * 16, 16)] = v - vi.astype(jnp.float32)
            return c1

        lax.fori_loop(0, NVREG, p1, 0)

        @pl.when(cid == 0)
        def _():
            copies = [
                pltpu.async_copy(
                    patch0_hbm.at[idx_v.at[pl.ds(g * 128, 128)]],
                    cor_v.at[pl.ds(g * 128, 128)],
                    gsem,
                )
                for g in range(B // 128)
            ]
            for cp in copies:
                cp.wait()

        @pl.when(cid == 1)
        def _():
            copies = [
                pltpu.async_copy(
                    patch1_hbm.at[idx_v.at[pl.ds(g * 128, 128)]],
                    cor_v.at[pl.ds(g * 128, 128)],
                    gsem,
                )
                for g in range(B // 128)
            ]
            for cp in copies:
                cp.wait()

        def p2(j, c2):
            rows = j * 16 + lanes
            fu = fu_v[pl.ds(j * 16, 16)]
            fv = fv_v[pl.ds(j * 16, 16)]
            w00 = (1.0 - fu) * (1.0 - fv)
            w01 = (1.0 - fu) * fv
            w10 = fu * (1.0 - fv)
            w11 = fu * fv
            key = []
            for ch in range(3):
                c00 = plsc.load_gather(cor_v, [rows, const16(ch)])
                c01 = plsc.load_gather(cor_v, [rows, const16(3 + ch)])
                c10 = plsc.load_gather(cor_v, [rows, const16(6 + ch)])
                c11 = plsc.load_gather(cor_v, [rows, const16(9 + ch)])
                key.append(w00 * c00 + w01 * c01 + w10 * c10 + w11 * c11)

            s = key[0] * float(RES_DN - 1)
            xi = jnp.minimum(s.astype(jnp.int32), RES_DN - 2)
            fx = s - xi.astype(jnp.float32)
            s = key[1] * float(RES_DN - 1)
            yi = jnp.minimum(s.astype(jnp.int32), RES_DN - 2)
            fy = s - yi.astype(jnp.float32)
            s = key[2] * float(RES_DN - 1)
            zi = jnp.minimum(s.astype(jnp.int32), RES_DN - 2)
            fz = s - zi.astype(jnp.float32)

            cell3 = ((xi * RES_DN + yi) * RES_DN + zi) * 3
            p00 = (1.0 - fx) * (1.0 - fy)
            p01 = (1.0 - fx) * fy
            p10 = fx * (1.0 - fy)
            p11 = fx * fy
            gz0 = 1.0 - fz
            w8 = [p00 * gz0, p00 * fz, p01 * gz0, p01 * fz,
                  p10 * gz0, p10 * fz, p11 * gz0, p11 * fz]
            offs = [0, 3, 60, 63, 1200, 1203, 1260, 1263]
            o_at = (rows >> 7) * 512 + (rows & 127)
            for ch in range(3):
                acc = w8[0] * plsc.load_gather(s0_v, [cell3 + ch])
                for kk in range(1, 8):
                    val = plsc.load_gather(s0_v, [cell3 + (offs[kk] + ch)])
                    acc = acc + w8[kk] * val
                plsc.store_scatter(out_v, [o_at + ch * 128], acc)
            return c2

        lax.fori_loop(0, NVREG, p2, 0)
        pltpu.sync_copy(out_v, out_hbm.at[pl.ds(4 * base, 4 * B)])
        return carry

    lax.fori_loop(0, NCHUNK, chunk_body, 0)


def kernel(x, grid1_table, grid0_table):
    one = x[0, 0] * 0.0 + 1.0
    xp = x.reshape(N_PTS // 128, 128, 2).transpose(0, 2, 1).reshape(-1)
    out, _, _ = _sc_net(xp, grid1_table.reshape(-1), grid0_table.reshape(-1))
    o3 = out.reshape(N_PTS // 128, 4, 128)[:, :3, :].transpose(0, 2, 1)
    return o3.reshape(N_PTS, 3) * one
